# baseline (speedup 1.0000x reference)
import functools

import jax
import jax.numpy as jnp
import numpy as np
from jax.experimental import pallas as pl
from jax.experimental.pallas import tpu as pltpu

N_SAMPLE = 8192
K = 32
IN_CH = 16
MLP_CHANNELS = [16, 32]
EPS = 1e-5

QB = 32
CHUNK = 128
S_MAX = 5


def _knn_body(q_ref, s_ref, sn_ref, o_ref, dd_ref):
    n = s_ref.shape[1]
    cn = n // CHUNK

    def dist():
        q = q_ref[...]
        qn = jnp.sum(q * q, axis=1, keepdims=True)
        d = jnp.dot(q, s_ref[...], preferred_element_type=jnp.float32)
        return (qn + sn_ref[...]) + d

    li = jax.lax.broadcasted_iota(jnp.int32, (QB, cn, CHUNK), 2)
    ci = jax.lax.broadcasted_iota(jnp.int32, (QB, cn), 1)
    dd_ref[...] = dist().reshape(QB, cn, CHUNK)
    d3 = dd_ref[...]
    bign = jnp.int32(n)

    cm = jnp.min(d3, axis=2)
    t_val = None
    for _ in range(K):
        m = jnp.min(cm, axis=1, keepdims=True)
        t_val = m
        cm = jnp.where(cm == m, jnp.inf, cm)
    thr = t_val[:, :, None]

    cand_v, cand_g = [], []
    am = jnp.full((QB, cn), -1, jnp.int32)
    for _ in range(S_MAX):
        nxt = jnp.min(
            jnp.where((d3 <= thr) & (li > am[:, :, None]), li, CHUNK), axis=2)
        v = jnp.min(jnp.where(li == nxt[:, :, None], d3, jnp.inf), axis=2)
        valid = nxt < CHUNK
        cand_v.append(jnp.where(valid, v, jnp.inf))
        cand_g.append(jnp.where(valid, ci * CHUNK + nxt, bign))
        am = nxt
    extra = jnp.min(
        jnp.where((d3 <= thr) & (li > am[:, :, None]), li, CHUNK), axis=2)
    flag = jnp.any(extra < CHUNK)

    V = jnp.concatenate(cand_v, axis=1)
    G = jnp.concatenate(cand_g, axis=1)
    for t in range(K):
        m = jnp.min(V, axis=1, keepdims=True)
        g = jnp.min(jnp.where(V == m, G, bign), axis=1, keepdims=True)
        o_ref[:, t] = g[:, 0]
        V = jnp.where((V == m) & (G == g), jnp.inf, V)

    @pl.when(flag)
    def _fallback():
        iota = jax.lax.broadcasted_iota(jnp.int32, (QB, n), 1)
        for t in range(K):
            dd2 = dd_ref[...].reshape(QB, n)
            mm = jnp.min(dd2, axis=1, keepdims=True)
            am = jnp.min(jnp.where(dd2 == mm, iota, bign), axis=1)
            o_ref[:, t] = am
            dd_ref[...] = jnp.where(iota == am[:, None], jnp.inf,
                                    dd2).reshape(QB, cn, CHUNK)


def _knn(support, query):
    n = support.shape[0]
    s = query.shape[0]
    sn = jnp.sum(support * support, axis=1)[None, :]
    s_aug = jnp.zeros((8, n), jnp.float32)
    s_aug = s_aug.at[0:3, :].set(-2.0 * support.T)
    q_aug = jnp.zeros((s, 8), jnp.float32)
    q_aug = q_aug.at[:, 0:3].set(query)
    return pl.pallas_call(
        _knn_body,
        grid=(s // QB,),
        in_specs=[
            pl.BlockSpec((QB, 8), lambda i: (i, 0)),
            pl.BlockSpec((8, n), lambda i: (0, 0)),
            pl.BlockSpec((1, n), lambda i: (0, 0)),
        ],
        out_specs=pl.BlockSpec((QB, K), lambda i: (i, 0)),
        out_shape=jax.ShapeDtypeStruct((s, K), jnp.int32),
        scratch_shapes=[pltpu.VMEM((QB, n // CHUNK, CHUNK), jnp.float32)],
    )(q_aug, s_aug, sn)


def _shared_mlp(x, W, gamma, beta):
    y = jnp.einsum('oi,gik->gok', W, x)
    mean = jnp.mean(y, axis=(0, 2), keepdims=True)
    var = jnp.mean((y - mean) ** 2, axis=(0, 2), keepdims=True)
    y = (y - mean) / jnp.sqrt(var + EPS)
    y = y * gamma[None, :, None] + beta[None, :, None]
    return jax.nn.relu(y)


def kernel(xyz, feats, W1, g1, b1, W2, g2, b2):
    B, C, N = feats.shape
    S = min(N_SAMPLE, N)
    perm = jax.random.permutation(jax.random.key(42), N)[:S]
    new_xyz = xyz[:, :, perm]
    support = xyz[0].T
    query = new_xyz[0].T
    idx = _knn(support, query)
    idx_t = idx.reshape(-1)
    gathered = feats[:, :, idx_t].reshape(B, C, S, K)
    rel_xyz = xyz[:, :, idx_t].reshape(B, 3, S, K) - new_xyz[:, :, :, None]
    group = jnp.concatenate([gathered, rel_xyz], axis=1)
    group = jnp.transpose(group, (0, 2, 1, 3)).reshape(B * S, C + 3, K)
    h = _shared_mlp(group, W1, g1, b1)
    h = _shared_mlp(h, W2, g2, b2)
    new_feats = jnp.max(h, axis=2).reshape(B, S, -1)
    new_feats = jnp.transpose(new_feats, (0, 2, 1))
    return (new_xyz, new_feats)


# speedup vs baseline: 1.8782x; 1.8782x over previous
import functools

import jax
import jax.numpy as jnp
import numpy as np
from jax.experimental import pallas as pl
from jax.experimental.pallas import tpu as pltpu

N_SAMPLE = 8192
K = 32
IN_CH = 16
MLP_CHANNELS = [16, 32]
EPS = 1e-5

QB = 128


def _knn_body(q_ref, s_ref, sn_ref, o_ref):
    n = s_ref.shape[1]
    q = q_ref[...]
    qn = jnp.sum(q * q, axis=1, keepdims=True)
    d = jnp.dot(q, s_ref[...], preferred_element_type=jnp.float32)
    d = (qn + sn_ref[...]) + d
    iota = jax.lax.broadcasted_iota(jnp.int32, (QB, n), 1)
    big = jnp.int32(n)
    for t in range(K):
        m = jnp.min(d, axis=1, keepdims=True)
        am = jnp.min(jnp.where(d == m, iota, big), axis=1)
        o_ref[:, t] = am
        d = jnp.where(iota == am[:, None], jnp.inf, d)


def _knn(support, query):
    n = support.shape[0]
    s = query.shape[0]
    sn = jnp.sum(support * support, axis=1)[None, :]
    s_aug = jnp.zeros((8, n), jnp.float32)
    s_aug = s_aug.at[0:3, :].set(-2.0 * support.T)
    q_aug = jnp.zeros((s, 8), jnp.float32)
    q_aug = q_aug.at[:, 0:3].set(query)
    return pl.pallas_call(
        _knn_body,
        grid=(s // QB,),
        in_specs=[
            pl.BlockSpec((QB, 8), lambda i: (i, 0)),
            pl.BlockSpec((8, n), lambda i: (0, 0)),
            pl.BlockSpec((1, n), lambda i: (0, 0)),
        ],
        out_specs=pl.BlockSpec((QB, K), lambda i: (i, 0)),
        out_shape=jax.ShapeDtypeStruct((s, K), jnp.int32),
    )(q_aug, s_aug, sn)


def _shared_mlp(x, W, gamma, beta):
    y = jnp.einsum('oi,gik->gok', W, x)
    mean = jnp.mean(y, axis=(0, 2), keepdims=True)
    var = jnp.mean((y - mean) ** 2, axis=(0, 2), keepdims=True)
    y = (y - mean) / jnp.sqrt(var + EPS)
    y = y * gamma[None, :, None] + beta[None, :, None]
    return jax.nn.relu(y)


def kernel(xyz, feats, W1, g1, b1, W2, g2, b2):
    B, C, N = feats.shape
    S = min(N_SAMPLE, N)
    perm = jax.random.permutation(jax.random.key(42), N)[:S]
    new_xyz = xyz[:, :, perm]
    support = xyz[0].T
    query = new_xyz[0].T
    idx = _knn(support, query)
    idx_t = idx.reshape(-1)
    gathered = feats[:, :, idx_t].reshape(B, C, S, K)
    rel_xyz = xyz[:, :, idx_t].reshape(B, 3, S, K) - new_xyz[:, :, :, None]
    group = jnp.concatenate([gathered, rel_xyz], axis=1)
    group = jnp.transpose(group, (0, 2, 1, 3)).reshape(B * S, C + 3, K)
    h = _shared_mlp(group, W1, g1, b1)
    h = _shared_mlp(h, W2, g2, b2)
    new_feats = jnp.max(h, axis=2).reshape(B, S, -1)
    new_feats = jnp.transpose(new_feats, (0, 2, 1))
    return (new_xyz, new_feats)
